# Initial kernel scaffold; baseline (speedup 1.0000x reference)
import jax
import jax.numpy as jnp
from jax.experimental import pallas as pl


def kernel(x, points, bounds, left_slopes, right_slopes):
    raise NotImplementedError("write your pallas kernel here")


# speedup vs baseline: 1861.9848x; 1861.9848x over previous
import jax
import jax.numpy as jnp
from jax.experimental import pallas as pl
from jax.experimental.pallas import tpu as pltpu


def _pwlu_kernel(points_ref, bounds_ref, ls_ref, rs_ref, x_ref, o_ref):
    p0 = points_ref[0, 0, 0]
    p1 = points_ref[0, 0, 1]
    p2 = points_ref[0, 0, 2]
    p3 = points_ref[0, 0, 3]
    p4 = points_ref[0, 0, 4]
    p5 = points_ref[0, 0, 5]
    p6 = points_ref[0, 0, 6]
    lb = bounds_ref[0, 0, 0]
    rb = bounds_ref[0, 0, 1]
    ls = ls_ref[0, 0, 0]
    rs = rs_ref[0, 0, 0]

    rl = rb - lb
    inv = 1.0 / rl
    sim_left = lb - rl

    f0 = p0 - ls * rl
    f1, f2, f3, f4, f5, f6, f7 = p0, p1, p2, p3, p4, p5, p6
    s0 = ls
    s1 = (p1 - p0) * inv
    s2 = (p2 - p1) * inv
    s3 = (p3 - p2) * inv
    s4 = (p4 - p3) * inv
    s5 = (p5 - p4) * inv
    s6 = (p6 - p5) * inv
    s7 = rs

    x = x_ref[0, 0]
    t = (x - sim_left) * inv
    r = jnp.floor(jnp.clip(t, 0.0, 7.007))
    d = t - r

    lt1 = r < 1.0
    lt2 = r < 2.0
    lt3 = r < 3.0
    lt5 = r < 5.0
    lt6 = r < 6.0
    lt7 = r < 7.0
    lt4 = r < 4.0

    fp = jnp.where(
        lt4,
        jnp.where(lt2, jnp.where(lt1, f0, f1), jnp.where(lt3, f2, f3)),
        jnp.where(lt6, jnp.where(lt5, f4, f5), jnp.where(lt7, f6, f7)),
    )
    sl = jnp.where(
        lt4,
        jnp.where(lt2, jnp.where(lt1, s0, s1), jnp.where(lt3, s2, s3)),
        jnp.where(lt6, jnp.where(lt5, s4, s5), jnp.where(lt7, s6, s7)),
    )
    o_ref[0, 0] = fp + d * sl


def kernel(x, points, bounds, left_slopes, right_slopes):
    B, C, H, W = x.shape
    n_points = points.shape[-1]
    pts3 = points.reshape(C, 1, n_points)
    bnd3 = bounds.reshape(C, 1, 2)
    ls3 = left_slopes.reshape(C, 1, 1)
    rs3 = right_slopes.reshape(C, 1, 1)

    grid = (B, C)
    smem = pltpu.SMEM
    return pl.pallas_call(
        _pwlu_kernel,
        grid=grid,
        in_specs=[
            pl.BlockSpec((1, 1, n_points), lambda b, c: (c, 0, 0), memory_space=smem),
            pl.BlockSpec((1, 1, 2), lambda b, c: (c, 0, 0), memory_space=smem),
            pl.BlockSpec((1, 1, 1), lambda b, c: (c, 0, 0), memory_space=smem),
            pl.BlockSpec((1, 1, 1), lambda b, c: (c, 0, 0), memory_space=smem),
            pl.BlockSpec((1, 1, H, W), lambda b, c: (b, c, 0, 0)),
        ],
        out_specs=pl.BlockSpec((1, 1, H, W), lambda b, c: (b, c, 0, 0)),
        out_shape=jax.ShapeDtypeStruct((B, C, H, W), x.dtype),
        compiler_params=pltpu.CompilerParams(
            dimension_semantics=("parallel", "parallel"),
        ),
    )(pts3, bnd3, ls3, rs3, x)
